# baseline (speedup 1.0000x reference)
import jax
import jax.numpy as jnp
from jax import lax
from jax.experimental import pallas as pl
from jax.experimental.pallas import tpu as pltpu
from jax.experimental.pallas import tpu_sc as plsc

B = 4096
R = 200
REL_DIM = 128
H = 128
V = 1024
NO_OP = 1000
RPAD = 1001
NEG = -1e10

BB = 256
BS = 512

SC_NC = 2
SC_NS = 16
NW = SC_NC * SC_NS
ROWS_PER_W = B // NW
SB = 32
N_SUB = ROWS_PER_W // SB
CHUNK_OFF = tuple(range(0, R - 16, 16)) + (R - 16,)
RSTRIDE = 256


def _dot(a, b, dims):
    return lax.dot_general(a, b, (dims, ((), ())),
                           preferred_element_type=jnp.float32)


def _scores_body(prev_ref, q_ref, tbl_ref, wih_ref, bih_ref, bhh_ref,
                 w1_ref, b1_ref, w2_ref, b2_ref, out_ref):
    prev = prev_ref[...]
    col = lax.broadcasted_iota(jnp.int32, (BB, V), 1)
    onehot = (prev == col).astype(jnp.bfloat16)
    tbl = tbl_ref[...]
    t_hi = tbl.astype(jnp.bfloat16)
    t_lo = (tbl - t_hi.astype(jnp.float32)).astype(jnp.bfloat16)
    prev_emb = (_dot(onehot, t_hi, ((1,), (0,))) +
                _dot(onehot, t_lo, ((1,), (0,))))
    gates = (_dot(prev_emb, wih_ref[...], ((1,), (1,)))
             + bih_ref[...] + bhh_ref[...])
    i_g = jax.nn.sigmoid(gates[:, 0 * H:1 * H])
    g_g = jnp.tanh(gates[:, 2 * H:3 * H])
    o_g = jax.nn.sigmoid(gates[:, 3 * H:4 * H])
    hx = o_g * jnp.tanh(i_g * g_g)
    lstm = jnp.where(prev == NO_OP, 0.0, hx)
    state = jnp.concatenate([lstm, q_ref[...]], axis=1)
    hidden = jax.nn.relu(_dot(state, w1_ref[...], ((1,), (1,))) + b1_ref[...])
    chosen = _dot(hidden, w2_ref[...], ((1,), (1,))) + b2_ref[...]
    scores = _dot(chosen, tbl, ((1,), (1,)))
    out_ref[...] = jnp.where(col == RPAD, NEG, scores).reshape(BB * V)


def _all_scores(prev2d, q, rel_table, wih, bih, bhh, w1, b1, w2, b2):
    grid = B // BB
    return pl.pallas_call(
        _scores_body,
        grid=(grid,),
        in_specs=[
            pl.BlockSpec((BB, 1), lambda i: (i, 0)),
            pl.BlockSpec((BB, REL_DIM), lambda i: (i, 0)),
            pl.BlockSpec((V, REL_DIM), lambda i: (0, 0)),
            pl.BlockSpec((4 * H, REL_DIM), lambda i: (0, 0)),
            pl.BlockSpec((1, 4 * H), lambda i: (0, 0)),
            pl.BlockSpec((1, 4 * H), lambda i: (0, 0)),
            pl.BlockSpec((256, 256), lambda i: (0, 0)),
            pl.BlockSpec((1, 256), lambda i: (0, 0)),
            pl.BlockSpec((REL_DIM, 256), lambda i: (0, 0)),
            pl.BlockSpec((1, REL_DIM), lambda i: (0, 0)),
        ],
        out_specs=pl.BlockSpec((BB * V,), lambda i: (i,)),
        out_shape=jax.ShapeDtypeStruct((B * V,), jnp.float32),
    )(prev2d, q, rel_table, wih, bih, bhh, w1, b1, w2, b2)


def _sc_gather_body(scores_hbm, idx_hbm, out_hbm,
                    sc0, sc1, ix0, ix1, ot0, ot1,
                    ss0, ss1, si0, si1, so0, so1):
    wid = lax.axis_index("s") * SC_NC + lax.axis_index("c")
    base = wid * ROWS_PER_W
    scb, ixb, otb = (sc0, sc1), (ix0, ix1), (ot0, ot1)
    ssem, isem, osem = (ss0, ss1), (si0, si1), (so0, so1)

    def start_in(bi, buf):
        row0 = base + bi * SB
        h_s = pltpu.async_copy(scores_hbm.at[pl.ds(row0 * V, SB * V)],
                               scb[buf], ssem[buf])
        h_i = pltpu.async_copy(idx_hbm.at[pl.ds(row0, SB)],
                               ixb[buf], isem[buf])
        return h_s, h_i

    in_handles = {0: start_in(0, 0)}
    out_handles = [None, None]
    for bi in range(N_SUB):
        buf = bi % 2
        h_s, h_i = in_handles.pop(bi)
        h_s.wait()
        h_i.wait()
        if bi + 1 < N_SUB:
            in_handles[bi + 1] = start_in(bi + 1, 1 - buf)
        if out_handles[buf] is not None:
            out_handles[buf].wait()
        sc_v, ix_v, ot_v = scb[buf], ixb[buf], otb[buf]

        def per_row(i, _):
            s_base = i * V
            o_base = i * RSTRIDE
            for off in CHUNK_OFF:
                cidx = ix_v[i, pl.ds(off, 16)]
                vals = plsc.load_gather(sc_v, [cidx + s_base])
                ot_v[pl.ds(o_base + off, 16)] = vals
            return _

        lax.fori_loop(0, SB, per_row, None)
        row0 = base + bi * SB
        out_handles[buf] = pltpu.async_copy(
            ot_v, out_hbm.at[pl.ds(row0 * RSTRIDE, SB * RSTRIDE)], osem[buf])
    for h in out_handles:
        if h is not None:
            h.wait()


def _sc_gather(scores_flat, idx):
    run = pl.kernel(
        _sc_gather_body,
        mesh=plsc.VectorSubcoreMesh(core_axis_name="c", subcore_axis_name="s"),
        compiler_params=pltpu.CompilerParams(needs_layout_passes=False),
        out_type=jax.ShapeDtypeStruct((B * RSTRIDE,), jnp.float32),
        scratch_types=[
            pltpu.VMEM((SB * V,), jnp.float32),
            pltpu.VMEM((SB * V,), jnp.float32),
            pltpu.VMEM((SB, R), jnp.int32),
            pltpu.VMEM((SB, R), jnp.int32),
            pltpu.VMEM((SB * RSTRIDE,), jnp.float32),
            pltpu.VMEM((SB * RSTRIDE,), jnp.float32),
            pltpu.SemaphoreType.DMA,
            pltpu.SemaphoreType.DMA,
            pltpu.SemaphoreType.DMA,
            pltpu.SemaphoreType.DMA,
            pltpu.SemaphoreType.DMA,
            pltpu.SemaphoreType.DMA,
        ],
    )
    return run(scores_flat, idx)


def _softmax_body(x_ref, o_ref):
    xw = x_ref[...].reshape(BS, RSTRIDE)
    col = lax.broadcasted_iota(jnp.int32, (BS, RSTRIDE), 1)
    valid = col < R
    m = jnp.max(jnp.where(valid, xw, NEG), axis=1, keepdims=True)
    s = jnp.sum(jnp.where(valid, jnp.exp(xw - m), 0.0), axis=1, keepdims=True)
    o_ref[...] = (xw - (m + jnp.log(s)))[:, :R]


def _log_softmax(sel_flat):
    grid = B // BS
    return pl.pallas_call(
        _softmax_body,
        grid=(grid,),
        in_specs=[pl.BlockSpec((BS * RSTRIDE,), lambda i: (i,))],
        out_specs=pl.BlockSpec((BS, R), lambda i: (i, 0)),
        out_shape=jax.ShapeDtypeStruct((B, R), jnp.float32),
    )(sel_flat)


def kernel(prev_relations, query_relation_embds, hl_space, rel_table,
           W_ih, W_hh, b_ih, b_hh, W1, b1, W2, b2):
    prev2d = prev_relations.astype(jnp.int32).reshape(B, 1)
    scores = _all_scores(prev2d, query_relation_embds, rel_table, W_ih,
                         b_ih.reshape(1, 4 * H), b_hh.reshape(1, 4 * H),
                         W1, b1.reshape(1, 256), W2, b2.reshape(1, REL_DIM))
    idx = hl_space[:, :, 0].astype(jnp.int32)
    sel = _sc_gather(scores, idx)
    return _log_softmax(sel)


# speedup vs baseline: 1.1400x; 1.1400x over previous
import jax
import jax.numpy as jnp
from jax import lax
from jax.experimental import pallas as pl
from jax.experimental.pallas import tpu as pltpu
from jax.experimental.pallas import tpu_sc as plsc

B = 4096
R = 200
REL_DIM = 128
H = 128
V = 1024
NO_OP = 1000
RPAD = 1001
NEG = -1e10

BB = 512
BS = 2048

SC_NC = 2
SC_NS = 16
NW = SC_NC * SC_NS
ROWS_PER_W = B // NW
SB = 32
N_SUB = ROWS_PER_W // SB
CHUNK_OFF = tuple(range(0, R - 16, 16)) + (R - 16,)
RSTRIDE = 256


def _dot(a, b, dims):
    return lax.dot_general(a, b, (dims, ((), ())),
                           preferred_element_type=jnp.float32)


def _scores_body(prev_ref, q_ref, tbl_ref, wih_ref, bih_ref, bhh_ref,
                 w1_ref, b1_ref, w2_ref, b2_ref, out_ref):
    prev = prev_ref[...]
    col = lax.broadcasted_iota(jnp.int32, (BB, V), 1)
    onehot = (prev == col).astype(jnp.bfloat16)
    tbl = tbl_ref[...]
    t_hi = tbl.astype(jnp.bfloat16)
    t_lo = (tbl - t_hi.astype(jnp.float32)).astype(jnp.bfloat16)
    prev_emb = (_dot(onehot, t_hi, ((1,), (0,))) +
                _dot(onehot, t_lo, ((1,), (0,))))
    gates = (_dot(prev_emb, wih_ref[...], ((1,), (1,)))
             + bih_ref[...] + bhh_ref[...])
    i_g = jax.nn.sigmoid(gates[:, 0 * H:1 * H])
    g_g = jnp.tanh(gates[:, 2 * H:3 * H])
    o_g = jax.nn.sigmoid(gates[:, 3 * H:4 * H])
    hx = o_g * jnp.tanh(i_g * g_g)
    lstm = jnp.where(prev == NO_OP, 0.0, hx)
    state = jnp.concatenate([lstm, q_ref[...]], axis=1)
    hidden = jax.nn.relu(_dot(state, w1_ref[...], ((1,), (1,))) + b1_ref[...])
    chosen = _dot(hidden, w2_ref[...], ((1,), (1,))) + b2_ref[...]
    scores = _dot(chosen, tbl, ((1,), (1,)))
    out_ref[...] = jnp.where(col == RPAD, NEG, scores).reshape(BB * V)


def _all_scores(prev2d, q, rel_table, wih, bih, bhh, w1, b1, w2, b2):
    grid = B // BB
    return pl.pallas_call(
        _scores_body,
        grid=(grid,),
        in_specs=[
            pl.BlockSpec((BB, 1), lambda i: (i, 0)),
            pl.BlockSpec((BB, REL_DIM), lambda i: (i, 0)),
            pl.BlockSpec((V, REL_DIM), lambda i: (0, 0)),
            pl.BlockSpec((4 * H, REL_DIM), lambda i: (0, 0)),
            pl.BlockSpec((1, 4 * H), lambda i: (0, 0)),
            pl.BlockSpec((1, 4 * H), lambda i: (0, 0)),
            pl.BlockSpec((256, 256), lambda i: (0, 0)),
            pl.BlockSpec((1, 256), lambda i: (0, 0)),
            pl.BlockSpec((REL_DIM, 256), lambda i: (0, 0)),
            pl.BlockSpec((1, REL_DIM), lambda i: (0, 0)),
        ],
        out_specs=pl.BlockSpec((BB * V,), lambda i: (i,)),
        out_shape=jax.ShapeDtypeStruct((B * V,), jnp.float32),
    )(prev2d, q, rel_table, wih, bih, bhh, w1, b1, w2, b2)


def _sc_gather_body(scores_hbm, idx_hbm, out_hbm,
                    sc0, sc1, ix0, ix1, ot0, ot1,
                    ss0, ss1, si0, si1, so0, so1):
    wid = lax.axis_index("s") * SC_NC + lax.axis_index("c")
    base = wid * ROWS_PER_W
    scb, ixb, otb = (sc0, sc1), (ix0, ix1), (ot0, ot1)
    ssem, isem, osem = (ss0, ss1), (si0, si1), (so0, so1)

    def start_in(bi, buf):
        row0 = base + bi * SB
        h_s = pltpu.async_copy(scores_hbm.at[pl.ds(row0 * V, SB * V)],
                               scb[buf], ssem[buf])
        h_i = pltpu.async_copy(idx_hbm.at[pl.ds(row0, SB)],
                               ixb[buf], isem[buf])
        return h_s, h_i

    in_handles = {0: start_in(0, 0)}
    out_handles = [None, None]
    for bi in range(N_SUB):
        buf = bi % 2
        h_s, h_i = in_handles.pop(bi)
        h_s.wait()
        h_i.wait()
        if bi + 1 < N_SUB:
            in_handles[bi + 1] = start_in(bi + 1, 1 - buf)
        if out_handles[buf] is not None:
            out_handles[buf].wait()
        sc_v, ix_v, ot_v = scb[buf], ixb[buf], otb[buf]

        def per_row(i, _):
            s_base = i * V
            for off in CHUNK_OFF:
                cidx = ix_v[i, pl.ds(off, 16)]
                vals = plsc.load_gather(sc_v, [cidx + s_base])
                ot_v[i, pl.ds(off, 16)] = vals
            return _

        lax.fori_loop(0, SB, per_row, None)
        row0 = base + bi * SB
        out_handles[buf] = pltpu.async_copy(
            ot_v, out_hbm.at[pl.ds(row0, SB)], osem[buf])
    for h in out_handles:
        if h is not None:
            h.wait()


def _sc_gather(scores_flat, idx):
    run = pl.kernel(
        _sc_gather_body,
        mesh=plsc.VectorSubcoreMesh(core_axis_name="c", subcore_axis_name="s"),
        compiler_params=pltpu.CompilerParams(needs_layout_passes=False),
        out_type=jax.ShapeDtypeStruct((B, R), jnp.float32),
        scratch_types=[
            pltpu.VMEM((SB * V,), jnp.float32),
            pltpu.VMEM((SB * V,), jnp.float32),
            pltpu.VMEM((SB, R), jnp.int32),
            pltpu.VMEM((SB, R), jnp.int32),
            pltpu.VMEM((SB, R), jnp.float32),
            pltpu.VMEM((SB, R), jnp.float32),
            pltpu.SemaphoreType.DMA,
            pltpu.SemaphoreType.DMA,
            pltpu.SemaphoreType.DMA,
            pltpu.SemaphoreType.DMA,
            pltpu.SemaphoreType.DMA,
            pltpu.SemaphoreType.DMA,
        ],
    )
    return run(scores_flat, idx)


def _softmax_body(x_ref, o_ref):
    x = x_ref[...]
    m = jnp.max(x, axis=1, keepdims=True)
    s = jnp.sum(jnp.exp(x - m), axis=1, keepdims=True)
    o_ref[...] = x - (m + jnp.log(s))


def _log_softmax(sel):
    grid = B // BS
    return pl.pallas_call(
        _softmax_body,
        grid=(grid,),
        in_specs=[pl.BlockSpec((BS, R), lambda i: (i, 0))],
        out_specs=pl.BlockSpec((BS, R), lambda i: (i, 0)),
        out_shape=jax.ShapeDtypeStruct((B, R), jnp.float32),
    )(sel)


def kernel(prev_relations, query_relation_embds, hl_space, rel_table,
           W_ih, W_hh, b_ih, b_hh, W1, b1, W2, b2):
    prev2d = prev_relations.astype(jnp.int32).reshape(B, 1)
    scores = _all_scores(prev2d, query_relation_embds, rel_table, W_ih,
                         b_ih.reshape(1, 4 * H), b_hh.reshape(1, 4 * H),
                         W1, b1.reshape(1, 256), W2, b2.reshape(1, REL_DIM))
    idx = hl_space[:, :, 0].astype(jnp.int32)
    sel = _sc_gather(scores, idx)
    return _log_softmax(sel)


# speedup vs baseline: 1.1440x; 1.0035x over previous
import jax
import jax.numpy as jnp
from jax import lax
from jax.experimental import pallas as pl
from jax.experimental.pallas import tpu as pltpu
from jax.experimental.pallas import tpu_sc as plsc

B = 4096
R = 200
REL_DIM = 128
H = 128
V = 1024
NO_OP = 1000
RPAD = 1001
NEG = -1e10

B2 = B // 2
BB = 512
BS = 2048

SC_NC = 2
SC_NS = 16
NW = SC_NC * SC_NS
ROWS_PER_W = B2 // NW
SB = 32
N_SUB = ROWS_PER_W // SB
CHUNK_OFF = tuple(range(0, R - 16, 16)) + (R - 16,)
RSTRIDE = 256


def _dot(a, b, dims):
    return lax.dot_general(a, b, (dims, ((), ())),
                           preferred_element_type=jnp.float32)


def _scores_body(prev_ref, q_ref, tbl_ref, wih_ref, bih_ref, bhh_ref,
                 w1_ref, b1_ref, w2_ref, b2_ref, out_ref):
    prev = prev_ref[...]
    col = lax.broadcasted_iota(jnp.int32, (BB, V), 1)
    onehot = (prev == col).astype(jnp.bfloat16)
    tbl = tbl_ref[...]
    t_hi = tbl.astype(jnp.bfloat16)
    t_lo = (tbl - t_hi.astype(jnp.float32)).astype(jnp.bfloat16)
    prev_emb = (_dot(onehot, t_hi, ((1,), (0,))) +
                _dot(onehot, t_lo, ((1,), (0,))))
    gates = (_dot(prev_emb, wih_ref[...], ((1,), (1,)))
             + bih_ref[...] + bhh_ref[...])
    i_g = jax.nn.sigmoid(gates[:, 0 * H:1 * H])
    g_g = jnp.tanh(gates[:, 2 * H:3 * H])
    o_g = jax.nn.sigmoid(gates[:, 3 * H:4 * H])
    hx = o_g * jnp.tanh(i_g * g_g)
    lstm = jnp.where(prev == NO_OP, 0.0, hx)
    state = jnp.concatenate([lstm, q_ref[...]], axis=1)
    hidden = jax.nn.relu(_dot(state, w1_ref[...], ((1,), (1,))) + b1_ref[...])
    chosen = _dot(hidden, w2_ref[...], ((1,), (1,))) + b2_ref[...]
    scores = _dot(chosen, tbl, ((1,), (1,)))
    out_ref[...] = jnp.where(col == RPAD, NEG, scores).reshape(BB * V)


def _all_scores(h, prev2d, q, rel_table, wih, bih, bhh, w1, b1, w2, b2):
    grid = B2 // BB
    goff = h * grid
    return pl.pallas_call(
        _scores_body,
        grid=(grid,),
        in_specs=[
            pl.BlockSpec((BB, 1), lambda i: (i + goff, 0)),
            pl.BlockSpec((BB, REL_DIM), lambda i: (i + goff, 0)),
            pl.BlockSpec((V, REL_DIM), lambda i: (0, 0)),
            pl.BlockSpec((4 * H, REL_DIM), lambda i: (0, 0)),
            pl.BlockSpec((1, 4 * H), lambda i: (0, 0)),
            pl.BlockSpec((1, 4 * H), lambda i: (0, 0)),
            pl.BlockSpec((256, 256), lambda i: (0, 0)),
            pl.BlockSpec((1, 256), lambda i: (0, 0)),
            pl.BlockSpec((REL_DIM, 256), lambda i: (0, 0)),
            pl.BlockSpec((1, REL_DIM), lambda i: (0, 0)),
        ],
        out_specs=pl.BlockSpec((BB * V,), lambda i: (i,)),
        out_shape=jax.ShapeDtypeStruct((B2 * V,), jnp.float32),
    )(prev2d, q, rel_table, wih, bih, bhh, w1, b1, w2, b2)


def _sc_gather_body(h, scores_hbm, idx_hbm, out_hbm,
                    sc0, sc1, ix0, ix1, ot0, ot1,
                    ss0, ss1, si0, si1, so0, so1):
    wid = lax.axis_index("s") * SC_NC + lax.axis_index("c")
    base = wid * ROWS_PER_W
    scb, ixb, otb = (sc0, sc1), (ix0, ix1), (ot0, ot1)
    ssem, isem, osem = (ss0, ss1), (si0, si1), (so0, so1)

    def start_in(bi, buf):
        row0 = base + bi * SB
        h_s = pltpu.async_copy(scores_hbm.at[pl.ds(row0 * V, SB * V)],
                               scb[buf], ssem[buf])
        h_i = pltpu.async_copy(idx_hbm.at[pl.ds(h * B2 + row0, SB)],
                               ixb[buf], isem[buf])
        return h_s, h_i

    in_handles = {0: start_in(0, 0)}
    out_handles = [None, None]
    for bi in range(N_SUB):
        buf = bi % 2
        h_s, h_i = in_handles.pop(bi)
        h_s.wait()
        h_i.wait()
        if bi + 1 < N_SUB:
            in_handles[bi + 1] = start_in(bi + 1, 1 - buf)
        if out_handles[buf] is not None:
            out_handles[buf].wait()
        sc_v, ix_v, ot_v = scb[buf], ixb[buf], otb[buf]

        def per_row(i, _):
            s_base = i * V
            for off in CHUNK_OFF:
                cidx = ix_v[i, pl.ds(off, 16)]
                vals = plsc.load_gather(sc_v, [cidx + s_base])
                ot_v[i, pl.ds(off, 16)] = vals
            return _

        lax.fori_loop(0, SB, per_row, None)
        row0 = base + bi * SB
        out_handles[buf] = pltpu.async_copy(
            ot_v, out_hbm.at[pl.ds(row0, SB)], osem[buf])
    for oh in out_handles:
        if oh is not None:
            oh.wait()


def _sc_gather(h, scores_flat, idx):
    import functools as _ft
    run = pl.kernel(
        _ft.partial(_sc_gather_body, h),
        mesh=plsc.VectorSubcoreMesh(core_axis_name="c", subcore_axis_name="s"),
        compiler_params=pltpu.CompilerParams(needs_layout_passes=False),
        out_type=jax.ShapeDtypeStruct((B2, R), jnp.float32),
        scratch_types=[
            pltpu.VMEM((SB * V,), jnp.float32),
            pltpu.VMEM((SB * V,), jnp.float32),
            pltpu.VMEM((SB, R), jnp.int32),
            pltpu.VMEM((SB, R), jnp.int32),
            pltpu.VMEM((SB, R), jnp.float32),
            pltpu.VMEM((SB, R), jnp.float32),
            pltpu.SemaphoreType.DMA,
            pltpu.SemaphoreType.DMA,
            pltpu.SemaphoreType.DMA,
            pltpu.SemaphoreType.DMA,
            pltpu.SemaphoreType.DMA,
            pltpu.SemaphoreType.DMA,
        ],
    )
    return run(scores_flat, idx)


def _softmax_body(x_ref, o_ref):
    x = x_ref[...]
    m = jnp.max(x, axis=1, keepdims=True)
    s = jnp.sum(jnp.exp(x - m), axis=1, keepdims=True)
    o_ref[...] = x - (m + jnp.log(s))


def _log_softmax(sel):
    grid = B2 // BS
    return pl.pallas_call(
        _softmax_body,
        grid=(grid,),
        in_specs=[pl.BlockSpec((BS, R), lambda i: (i, 0))],
        out_specs=pl.BlockSpec((BS, R), lambda i: (i, 0)),
        out_shape=jax.ShapeDtypeStruct((B2, R), jnp.float32),
    )(sel)


def kernel(prev_relations, query_relation_embds, hl_space, rel_table,
           W_ih, W_hh, b_ih, b_hh, W1, b1, W2, b2):
    prev2d = prev_relations.astype(jnp.int32).reshape(B, 1)
    idx = hl_space[:, :, 0].astype(jnp.int32)
    args = (query_relation_embds, rel_table, W_ih,
            b_ih.reshape(1, 4 * H), b_hh.reshape(1, 4 * H),
            W1, b1.reshape(1, 256), W2, b2.reshape(1, REL_DIM))
    scores0 = _all_scores(0, prev2d, *args)
    sel0 = _sc_gather(0, scores0, idx)
    scores1 = _all_scores(1, prev2d, *args)
    sel1 = _sc_gather(1, scores1, idx)
    out0 = _log_softmax(sel0)
    out1 = _log_softmax(sel1)
    return jnp.concatenate([out0, out1], axis=0)
